# Initial kernel scaffold; baseline (speedup 1.0000x reference)
import jax
import jax.numpy as jnp
from jax.experimental import pallas as pl


def kernel(x, edge_index, edge_type, basis1, comp1, root1, bias1, basis2, comp2, root2, bias2):
    raise NotImplementedError("write your pallas kernel here")


# speedup vs baseline: 27.5701x; 27.5701x over previous
import jax
import jax.numpy as jnp
from jax import lax
from jax.experimental import pallas as pl
from jax.experimental.pallas import tpu as pltpu
from jax.experimental.pallas import tpu_sc as plsc

_N = 10000
_E = 320000
_D = 128
_H = 64
_O = 4
_R = 3

_NPAD = 10112
_SN = _R * _NPAD
_K = 80
_STRIPE = _SN // 16

_EPT = _E // 16
_NCH1 = _EPT // _K
_EPW = _E // 32
_NCH2 = _EPW // _K

_BLK = 1000
_HW = _H // 2

_mesh = plsc.VectorSubcoreMesh(core_axis_name="c", subcore_axis_name="s")
_sc_params = pltpu.CompilerParams(use_tc_tiling_on_sc=False)


def _mm_body(x_ref, w_ref, b_ref, o_ref):
    o_ref[...] = (
        jnp.dot(x_ref[...], w_ref[...], preferred_element_type=jnp.float32)
        + b_ref[...]
    )


def _edge_prep_body(d_ref, t_ref, o_ref):
    o_ref[...] = t_ref[...] * _NPAD + d_ref[...]


def _seg1_body(src_hbm, sidx_hbm, y_hbm, za_hbm, zc_hbm, aout_hbm, cout_hbm,
               a_sh, c_sh, src_v, sidx_v, rows_v, ones_v, sem):
    cid = lax.axis_index("c")
    sid = lax.axis_index("s")
    base = sid * _STRIPE

    pltpu.sync_copy(za_hbm, a_sh.at[pl.ds(base, _STRIPE)])
    pltpu.sync_copy(zc_hbm, c_sh.at[pl.ds(base, _STRIPE)])
    ones16 = jnp.ones((16,), jnp.float32)
    for q in range(_K // 16):
        ones_v[pl.ds(q * 16, 16)] = ones16

    pltpu.sync_copy(src_hbm.at[sid], src_v)
    pltpu.sync_copy(sidx_hbm.at[sid], sidx_v)
    plsc.subcore_barrier()

    do_counts = cid == 0

    def chunk(ci, _):
        pltpu.async_copy(y_hbm.at[cid].at[src_v.at[ci]], rows_v, sem).wait()
        pltpu.sync_copy(rows_v, a_sh.at[sidx_v.at[ci]], add=True)

        @pl.when(do_counts)
        def _():
            pltpu.sync_copy(ones_v, c_sh.at[sidx_v.at[ci]], add=True)
        return 0
    lax.fori_loop(0, _NCH1, chunk, 0)

    plsc.subcore_barrier()
    pltpu.sync_copy(a_sh.at[pl.ds(base, _STRIPE)],
                    aout_hbm.at[cid, pl.ds(base, _STRIPE)])

    @pl.when(do_counts)
    def _():
        pltpu.sync_copy(c_sh.at[pl.ds(base, _STRIPE)],
                        cout_hbm.at[pl.ds(base, _STRIPE)])


def _seg2_body(src_hbm, sidx_hbm, y_hbm, za_hbm, aout_hbm,
               a_sh, src_v, sidx_v, rows_v, sem):
    cid = lax.axis_index("c")
    sid = lax.axis_index("s")
    wid = sid * 2 + cid
    base = sid * _STRIPE

    pltpu.sync_copy(za_hbm, a_sh.at[pl.ds(base, _STRIPE)])
    pltpu.sync_copy(src_hbm.at[wid], src_v)
    pltpu.sync_copy(sidx_hbm.at[wid], sidx_v)
    plsc.subcore_barrier()

    def chunk(ci, _):
        pltpu.async_copy(y_hbm.at[src_v.at[ci]], rows_v, sem).wait()
        pltpu.sync_copy(rows_v, a_sh.at[sidx_v.at[ci]], add=True)
        return 0
    lax.fori_loop(0, _NCH2, chunk, 0)

    plsc.subcore_barrier()
    pltpu.sync_copy(a_sh.at[pl.ds(base, _STRIPE)],
                    aout_hbm.at[cid, pl.ds(base, _STRIPE)])


_seg1 = pl.kernel(
    _seg1_body,
    out_type=(
        jax.ShapeDtypeStruct((2, _SN, _HW), jnp.float32),
        jax.ShapeDtypeStruct((_SN,), jnp.float32),
    ),
    mesh=_mesh,
    scratch_types=[
        pltpu.VMEM_SHARED((_SN, _HW), jnp.float32),
        pltpu.VMEM_SHARED((_SN,), jnp.float32),
        pltpu.VMEM((_NCH1, _K), jnp.int32),
        pltpu.VMEM((_NCH1, _K), jnp.int32),
        pltpu.VMEM((_K, _HW), jnp.float32),
        pltpu.VMEM((_K,), jnp.float32),
        pltpu.SemaphoreType.DMA,
    ],
    compiler_params=_sc_params,
)

_seg2 = pl.kernel(
    _seg2_body,
    out_type=jax.ShapeDtypeStruct((2, _SN, 2 * _O), jnp.float32),
    mesh=_mesh,
    scratch_types=[
        pltpu.VMEM_SHARED((_SN, 2 * _O), jnp.float32),
        pltpu.VMEM((_NCH2, _K), jnp.int32),
        pltpu.VMEM((_NCH2, _K), jnp.int32),
        pltpu.VMEM((_K, 2 * _O), jnp.float32),
        pltpu.SemaphoreType.DMA,
    ],
    compiler_params=_sc_params,
)


def _comb1_body(y_ref, a_ref, c_ref, comp_ref, w_ref, b_ref, o_ref):
    r1 = y_ref[:, _H:]
    acc = jnp.zeros((_BLK, _H), jnp.float32)
    for r in range(_R):
        s = jnp.concatenate([a_ref[0, r], a_ref[1, r]], axis=1)
        acc = acc + comp_ref[r, 0] * s / jnp.maximum(c_ref[r], 1.0)
    h = jnp.maximum(r1 + acc, 0.0)
    o_ref[...] = (
        jnp.dot(h, w_ref[...], preferred_element_type=jnp.float32) + b_ref[...]
    )


def _comb2_body(y_ref, a_ref, c_ref, comp_ref, o_ref):
    r2 = y_ref[:, _O:]
    acc = jnp.zeros((_BLK, _O), jnp.float32)
    for r in range(_R):
        s = a_ref[0, r, :, :_O] + a_ref[1, r, :, :_O]
        acc = acc + comp_ref[r, 0] * s / jnp.maximum(c_ref[r], 1.0)
    z = r2 + acc
    z = z - jnp.max(z, axis=1, keepdims=True)
    ez = jnp.exp(z)
    o_ref[...] = ez / jnp.sum(ez, axis=1, keepdims=True)


def kernel(x, edge_index, edge_type, basis1, comp1, root1, bias1,
           basis2, comp2, root2, bias2):
    src = edge_index[0].astype(jnp.int32)
    dst = edge_index[1].astype(jnp.int32)
    et = edge_type.astype(jnp.int32)

    sidx2d = pl.pallas_call(
        _edge_prep_body,
        out_shape=jax.ShapeDtypeStruct((_E // 128, 128), jnp.int32),
    )(dst.reshape(_E // 128, 128), et.reshape(_E // 128, 128))

    src16 = src.reshape(16, _NCH1, _K)
    sidx16 = sidx2d.reshape(16, _NCH1, _K)
    src32 = src.reshape(32, _NCH2, _K)
    sidx32 = sidx2d.reshape(32, _NCH2, _K)

    w1cat = jnp.concatenate([basis1[0], root1], axis=1)
    b1cat = jnp.concatenate([jnp.zeros((_H,), jnp.float32), bias1])[None, :]
    y1r = pl.pallas_call(
        _mm_body,
        grid=(_N // _BLK,),
        in_specs=[
            pl.BlockSpec((_BLK, _D), lambda i: (i, 0)),
            pl.BlockSpec((_D, 2 * _H), lambda i: (0, 0)),
            pl.BlockSpec((1, 2 * _H), lambda i: (0, 0)),
        ],
        out_specs=pl.BlockSpec((_BLK, 2 * _H), lambda i: (i, 0)),
        out_shape=jax.ShapeDtypeStruct((_N, 2 * _H), jnp.float32),
    )(x, w1cat, b1cat)

    y1s = y1r[:, :_H].reshape(_N, 2, _HW).transpose(1, 0, 2)

    za1 = jnp.zeros((_STRIPE, _HW), jnp.float32)
    zc = jnp.zeros((_STRIPE,), jnp.float32)
    a1p, c1 = _seg1(src16, sidx16, y1s, za1, zc)
    a1 = a1p.reshape(2, _R, _NPAD, _HW)
    c4 = c1.reshape(_R, _NPAD, 1)

    w2cat = jnp.concatenate([basis2[0], root2], axis=1)
    b2cat = jnp.concatenate([jnp.zeros((_O,), jnp.float32), bias2])[None, :]
    y2r = pl.pallas_call(
        _comb1_body,
        grid=(_N // _BLK,),
        in_specs=[
            pl.BlockSpec((_BLK, 2 * _H), lambda i: (i, 0)),
            pl.BlockSpec((2, _R, _BLK, _HW), lambda i: (0, 0, i, 0)),
            pl.BlockSpec((_R, _BLK, 1), lambda i: (0, i, 0)),
            pl.BlockSpec((_R, 1), lambda i: (0, 0)),
            pl.BlockSpec((_H, 2 * _O), lambda i: (0, 0)),
            pl.BlockSpec((1, 2 * _O), lambda i: (0, 0)),
        ],
        out_specs=pl.BlockSpec((_BLK, 2 * _O), lambda i: (i, 0)),
        out_shape=jax.ShapeDtypeStruct((_N, 2 * _O), jnp.float32),
    )(y1r, a1, c4, comp1, w2cat, b2cat)

    za2 = jnp.zeros((_STRIPE, 2 * _O), jnp.float32)
    a2p = _seg2(src32, sidx32, y2r, za2)
    a2 = a2p.reshape(2, _R, _NPAD, 2 * _O)

    out = pl.pallas_call(
        _comb2_body,
        grid=(_N // _BLK,),
        in_specs=[
            pl.BlockSpec((_BLK, 2 * _O), lambda i: (i, 0)),
            pl.BlockSpec((2, _R, _BLK, 2 * _O), lambda i: (0, 0, i, 0)),
            pl.BlockSpec((_R, _BLK, 1), lambda i: (0, i, 0)),
            pl.BlockSpec((_R, 1), lambda i: (0, 0)),
        ],
        out_specs=pl.BlockSpec((_BLK, _O), lambda i: (i, 0)),
        out_shape=jax.ShapeDtypeStruct((_N, _O), jnp.float32),
    )(y2r, a2, c4, comp2)

    return out


# speedup vs baseline: 39.7963x; 1.4435x over previous
import jax
import jax.numpy as jnp
from jax import lax
from jax.experimental import pallas as pl
from jax.experimental.pallas import tpu as pltpu
from jax.experimental.pallas import tpu_sc as plsc

_N = 10000
_E = 320000
_D = 128
_H = 64
_O = 4
_R = 3

_NPAD = 10112
_SN = _R * _NPAD
_K = 80
_STRIPE = _SN // 16

_EPT = _E // 16
_NCH1 = _EPT // _K
_EPW = _E // 32
_NCH2 = _EPW // _K

_BLK = 1000
_HW = _H // 2

_mesh = plsc.VectorSubcoreMesh(core_axis_name="c", subcore_axis_name="s")
_sc_params = pltpu.CompilerParams(use_tc_tiling_on_sc=False)


def _mm_body(x_ref, w_ref, b_ref, o_ref, o2_ref):
    y = (
        jnp.dot(x_ref[...], w_ref[...], preferred_element_type=jnp.float32)
        + b_ref[...]
    )
    o_ref[...] = y
    o2_ref[0] = y[:, :_HW]
    o2_ref[1] = y[:, _HW:_H]


def _edge_prep_body(d_ref, t_ref, o_ref):
    o_ref[...] = t_ref[...] * _NPAD + d_ref[...]


def _seg1_body(src_hbm, sidx_hbm, y_hbm, za_hbm, zc_hbm, aout_hbm, cout_hbm,
               a_sh, c_sh, src_v, sidx_v, rows0_v, rows1_v, ones_v,
               sem0, sem1):
    cid = lax.axis_index("c")
    sid = lax.axis_index("s")
    base = sid * _STRIPE

    pltpu.sync_copy(za_hbm, a_sh.at[pl.ds(base, _STRIPE)])
    pltpu.sync_copy(zc_hbm, c_sh.at[pl.ds(base, _STRIPE)])
    ones16 = jnp.ones((16,), jnp.float32)
    for q in range(_K // 16):
        ones_v[pl.ds(q * 16, 16)] = ones16

    pltpu.sync_copy(src_hbm.at[sid], src_v)
    pltpu.sync_copy(sidx_hbm.at[sid], sidx_v)
    plsc.subcore_barrier()

    do_counts = cid == 0
    ysrc = y_hbm.at[cid]

    def gather(ci, buf, sem):
        pltpu.async_copy(ysrc.at[src_v.at[ci]], buf, sem)

    def drain_scatter(ci, buf, sem):
        pltpu.make_async_copy(ysrc.at[src_v.at[ci]], buf, sem).wait()
        pltpu.sync_copy(buf, a_sh.at[sidx_v.at[ci]], add=True)

        @pl.when(do_counts)
        def _():
            pltpu.sync_copy(ones_v, c_sh.at[sidx_v.at[ci]], add=True)

    gather(0, rows0_v, sem0)

    def pair(g, _):
        c0 = 2 * g
        gather(c0 + 1, rows1_v, sem1)
        drain_scatter(c0, rows0_v, sem0)

        @pl.when(c0 + 2 < _NCH1)
        def _():
            gather(c0 + 2, rows0_v, sem0)
        drain_scatter(c0 + 1, rows1_v, sem1)
        return 0
    lax.fori_loop(0, _NCH1 // 2, pair, 0)
    if _NCH1 % 2:
        drain_scatter(_NCH1 - 1, rows0_v, sem0)

    plsc.subcore_barrier()
    pltpu.sync_copy(a_sh.at[pl.ds(base, _STRIPE)],
                    aout_hbm.at[cid, pl.ds(base, _STRIPE)])

    @pl.when(do_counts)
    def _():
        pltpu.sync_copy(c_sh.at[pl.ds(base, _STRIPE)],
                        cout_hbm.at[pl.ds(base, _STRIPE)])


def _seg2_body(src_hbm, sidx_hbm, y_hbm, za_hbm, aout_hbm,
               a_sh, src_v, sidx_v, rows0_v, rows1_v, sem0, sem1):
    cid = lax.axis_index("c")
    sid = lax.axis_index("s")
    wid = sid * 2 + cid
    base = sid * _STRIPE

    pltpu.sync_copy(za_hbm, a_sh.at[pl.ds(base, _STRIPE)])
    pltpu.sync_copy(src_hbm.at[wid], src_v)
    pltpu.sync_copy(sidx_hbm.at[wid], sidx_v)
    plsc.subcore_barrier()

    def gather(ci, buf, sem):
        pltpu.async_copy(y_hbm.at[src_v.at[ci]], buf, sem)

    def drain_scatter(ci, buf, sem):
        pltpu.make_async_copy(y_hbm.at[src_v.at[ci]], buf, sem).wait()
        pltpu.sync_copy(buf, a_sh.at[sidx_v.at[ci]], add=True)

    gather(0, rows0_v, sem0)

    def pair(g, _):
        c0 = 2 * g
        gather(c0 + 1, rows1_v, sem1)
        drain_scatter(c0, rows0_v, sem0)

        @pl.when(c0 + 2 < _NCH2)
        def _():
            gather(c0 + 2, rows0_v, sem0)
        drain_scatter(c0 + 1, rows1_v, sem1)
        return 0
    lax.fori_loop(0, _NCH2 // 2, pair, 0)
    if _NCH2 % 2:
        drain_scatter(_NCH2 - 1, rows0_v, sem0)

    plsc.subcore_barrier()
    pltpu.sync_copy(a_sh.at[pl.ds(base, _STRIPE)],
                    aout_hbm.at[cid, pl.ds(base, _STRIPE)])


_seg1 = pl.kernel(
    _seg1_body,
    out_type=(
        jax.ShapeDtypeStruct((2, _SN, _HW), jnp.float32),
        jax.ShapeDtypeStruct((_SN,), jnp.float32),
    ),
    mesh=_mesh,
    scratch_types=[
        pltpu.VMEM_SHARED((_SN, _HW), jnp.float32),
        pltpu.VMEM_SHARED((_SN,), jnp.float32),
        pltpu.VMEM((_NCH1, _K), jnp.int32),
        pltpu.VMEM((_NCH1, _K), jnp.int32),
        pltpu.VMEM((_K, _HW), jnp.float32),
        pltpu.VMEM((_K, _HW), jnp.float32),
        pltpu.VMEM((_K,), jnp.float32),
        pltpu.SemaphoreType.DMA,
        pltpu.SemaphoreType.DMA,
    ],
    compiler_params=_sc_params,
)

_seg2 = pl.kernel(
    _seg2_body,
    out_type=jax.ShapeDtypeStruct((2, _SN, 2 * _O), jnp.float32),
    mesh=_mesh,
    scratch_types=[
        pltpu.VMEM_SHARED((_SN, 2 * _O), jnp.float32),
        pltpu.VMEM((_NCH2, _K), jnp.int32),
        pltpu.VMEM((_NCH2, _K), jnp.int32),
        pltpu.VMEM((_K, 2 * _O), jnp.float32),
        pltpu.VMEM((_K, 2 * _O), jnp.float32),
        pltpu.SemaphoreType.DMA,
        pltpu.SemaphoreType.DMA,
    ],
    compiler_params=_sc_params,
)


def _comb1_body(y_ref, a_ref, c_ref, comp_ref, w_ref, b_ref, o_ref):
    r1 = y_ref[:, _H:]
    acc = jnp.zeros((_BLK, _H), jnp.float32)
    for r in range(_R):
        s = jnp.concatenate([a_ref[0, r], a_ref[1, r]], axis=1)
        acc = acc + comp_ref[r, 0] * s / jnp.maximum(c_ref[r], 1.0)
    h = jnp.maximum(r1 + acc, 0.0)
    o_ref[...] = (
        jnp.dot(h, w_ref[...], preferred_element_type=jnp.float32) + b_ref[...]
    )


def _comb2_body(y_ref, a_ref, c_ref, comp_ref, o_ref):
    r2 = y_ref[:, _O:]
    acc = jnp.zeros((_BLK, _O), jnp.float32)
    for r in range(_R):
        s = a_ref[0, r, :, :_O] + a_ref[1, r, :, :_O]
        acc = acc + comp_ref[r, 0] * s / jnp.maximum(c_ref[r], 1.0)
    z = r2 + acc
    z = z - jnp.max(z, axis=1, keepdims=True)
    ez = jnp.exp(z)
    o_ref[...] = ez / jnp.sum(ez, axis=1, keepdims=True)


def kernel(x, edge_index, edge_type, basis1, comp1, root1, bias1,
           basis2, comp2, root2, bias2):
    src = edge_index[0].astype(jnp.int32)
    dst = edge_index[1].astype(jnp.int32)
    et = edge_type.astype(jnp.int32)

    sidx2d = pl.pallas_call(
        _edge_prep_body,
        out_shape=jax.ShapeDtypeStruct((_E // 128, 128), jnp.int32),
    )(dst.reshape(_E // 128, 128), et.reshape(_E // 128, 128))

    src16 = src.reshape(16, _NCH1, _K)
    sidx16 = sidx2d.reshape(16, _NCH1, _K)
    src32 = src.reshape(32, _NCH2, _K)
    sidx32 = sidx2d.reshape(32, _NCH2, _K)

    w1cat = jnp.concatenate([basis1[0], root1], axis=1)
    b1cat = jnp.concatenate([jnp.zeros((_H,), jnp.float32), bias1])[None, :]
    y1r, y1s = pl.pallas_call(
        _mm_body,
        grid=(_N // _BLK,),
        in_specs=[
            pl.BlockSpec((_BLK, _D), lambda i: (i, 0)),
            pl.BlockSpec((_D, 2 * _H), lambda i: (0, 0)),
            pl.BlockSpec((1, 2 * _H), lambda i: (0, 0)),
        ],
        out_specs=[
            pl.BlockSpec((_BLK, 2 * _H), lambda i: (i, 0)),
            pl.BlockSpec((2, _BLK, _HW), lambda i: (0, i, 0)),
        ],
        out_shape=[
            jax.ShapeDtypeStruct((_N, 2 * _H), jnp.float32),
            jax.ShapeDtypeStruct((2, _N, _HW), jnp.float32),
        ],
    )(x, w1cat, b1cat)

    za1 = jnp.zeros((_STRIPE, _HW), jnp.float32)
    zc = jnp.zeros((_STRIPE,), jnp.float32)
    a1p, c1 = _seg1(src16, sidx16, y1s, za1, zc)
    a1 = a1p.reshape(2, _R, _NPAD, _HW)
    c4 = c1.reshape(_R, _NPAD, 1)

    w2cat = jnp.concatenate([basis2[0], root2], axis=1)
    b2cat = jnp.concatenate([jnp.zeros((_O,), jnp.float32), bias2])[None, :]
    y2r = pl.pallas_call(
        _comb1_body,
        grid=(_N // _BLK,),
        in_specs=[
            pl.BlockSpec((_BLK, 2 * _H), lambda i: (i, 0)),
            pl.BlockSpec((2, _R, _BLK, _HW), lambda i: (0, 0, i, 0)),
            pl.BlockSpec((_R, _BLK, 1), lambda i: (0, i, 0)),
            pl.BlockSpec((_R, 1), lambda i: (0, 0)),
            pl.BlockSpec((_H, 2 * _O), lambda i: (0, 0)),
            pl.BlockSpec((1, 2 * _O), lambda i: (0, 0)),
        ],
        out_specs=pl.BlockSpec((_BLK, 2 * _O), lambda i: (i, 0)),
        out_shape=jax.ShapeDtypeStruct((_N, 2 * _O), jnp.float32),
    )(y1r, a1, c4, comp1, w2cat, b2cat)

    za2 = jnp.zeros((_STRIPE, 2 * _O), jnp.float32)
    a2p = _seg2(src32, sidx32, y2r, za2)
    a2 = a2p.reshape(2, _R, _NPAD, 2 * _O)

    out = pl.pallas_call(
        _comb2_body,
        grid=(_N // _BLK,),
        in_specs=[
            pl.BlockSpec((_BLK, 2 * _O), lambda i: (i, 0)),
            pl.BlockSpec((2, _R, _BLK, 2 * _O), lambda i: (0, 0, i, 0)),
            pl.BlockSpec((_R, _BLK, 1), lambda i: (0, i, 0)),
            pl.BlockSpec((_R, 1), lambda i: (0, 0)),
        ],
        out_specs=pl.BlockSpec((_BLK, _O), lambda i: (i, 0)),
        out_shape=jax.ShapeDtypeStruct((_N, _O), jnp.float32),
    )(y2r, a2, c4, comp2)

    return out
